# baseline (speedup 1.0000x reference)
import jax
import jax.numpy as jnp
from jax import lax
from jax.experimental import pallas as pl
from jax.experimental.pallas import tpu as pltpu
from jax.experimental.pallas import tpu_sc as plsc

_NC = 2
_NS = 16
_B = 128
_R = 400


def _round_up(v, m):
  return (v + m - 1) // m * m


_NSLOT = 2
_C = 16


def _sage_aggregate(x, src2, dst2, n_pad):
  n, d = x.shape
  nb_total = src2.shape[0]
  nw = _NC * _NS
  nb = nb_total // nw
  rows_per_tile = n_pad // _NS
  nchunk = nb // _C
  assert nb_total % nw == 0 and nb % _C == 0 and nchunk >= 2
  assert rows_per_tile % _B == 0

  mesh = plsc.VectorSubcoreMesh(core_axis_name="c", subcore_axis_name="s")

  def body(x_hbm, src_hbm, dst_hbm, sum_hbm, cnt_hbm,
           sv, dv, r0, r1, ones_v, zrow_v, acc_sh, cnt_sh,
           g0, g1, s0, s1, sc):
    rows = (r0, r1)
    sem_g = (g0, g1)
    sem_s = (s0, s1)
    c = lax.axis_index("c")
    s = lax.axis_index("s")
    w = c * _NS + s
    row0 = s * rows_per_tile
    tile_b0 = w * nb

    @pl.loop(0, _B)
    def _(r):
      @pl.loop(0, d, step=16)
      def _(k):
        r0[r, pl.ds(k, 16)] = jnp.zeros((16,), jnp.float32)

    @pl.loop(0, rows_per_tile, step=16)
    def _(i):
      zrow_v[pl.ds(i, 16)] = jnp.zeros((16,), jnp.float32)

    @pl.loop(0, _B, step=16)
    def _(i):
      ones_v[pl.ds(i, 16)] = jnp.ones((16,), jnp.float32)

    def load_chunk(cc):
      pltpu.sync_copy(src_hbm.at[pl.ds(tile_b0 + cc * _C, _C)], sv.at[cc % 2])
      pltpu.sync_copy(dst_hbm.at[pl.ds(tile_b0 + cc * _C, _C)], dv.at[cc % 2])

    load_chunk(0)
    load_chunk(1)

    for j in range(rows_per_tile // _B):
      pltpu.sync_copy(r0, acc_sh.at[pl.ds(row0 + j * _B, _B)])
    pltpu.sync_copy(zrow_v, cnt_sh.at[pl.ds(row0, rows_per_tile)])
    plsc.subcore_barrier()

    def idx(buf, b):
      return buf.at[(b // _C) % 2, b % _C]

    def fire_g(j, b):
      pltpu.async_copy(x_hbm.at[idx(sv, b)], rows[j], sem_g[j])

    def wait_g(j):
      pltpu.make_async_copy(x_hbm.at[sv.at[0, 0]], rows[j], sem_g[j]).wait()

    def fire_s(j, b):
      pltpu.async_copy(rows[j], acc_sh.at[idx(dv, b)], sem_s[j], add=True)
      pltpu.async_copy(ones_v, cnt_sh.at[idx(dv, b)], sc, add=True)

    def wait_s(j):
      pltpu.make_async_copy(rows[j], acc_sh.at[dv.at[0, 0]], sem_s[j]).wait()

    fire_g(0, 0)
    fire_g(1, 1)
    wait_g(0)
    fire_s(0, 0)
    wait_g(1)
    fire_s(1, 1)
    wait_s(0)
    fire_g(0, 2)

    @pl.loop(1, nb // 2 - 1)
    def _(k):
      for j in range(2):
        b = 2 * k + j
        wait_g(j)
        fire_s(j, b)
        wait_s(1 - j)
        @pl.when(jnp.logical_and(b % _C == 0, b + _C < nb))
        def _():
          cc1 = b // _C + 1
          pltpu.sync_copy(src_hbm.at[pl.ds(tile_b0 + cc1 * _C, _C)],
                          sv.at[cc1 % 2])
          pltpu.sync_copy(dst_hbm.at[pl.ds(tile_b0 + cc1 * _C, _C)],
                          dv.at[cc1 % 2])
        fire_g(1 - j, b + 1)

    wait_g(0)
    fire_s(0, nb - 2)
    wait_s(1)
    fire_g(1, nb - 1)
    wait_g(1)
    fire_s(1, nb - 1)
    wait_s(0)
    wait_s(1)

    @pl.loop(0, nb)
    def _(i):
      pltpu.make_async_copy(ones_v, cnt_sh.at[dv.at[0, 0]], sc).wait()

    plsc.subcore_barrier()

    pltpu.sync_copy(acc_sh.at[pl.ds(row0, rows_per_tile)],
                    sum_hbm.at[c, pl.ds(row0, rows_per_tile)])
    pltpu.sync_copy(cnt_sh.at[pl.ds(row0, rows_per_tile)],
                    cnt_hbm.at[pl.ds(c * n_pad + row0, rows_per_tile)])

  kern = pl.kernel(
      body,
      out_type=[
          jax.ShapeDtypeStruct((_NC, n_pad, d), jnp.float32),
          jax.ShapeDtypeStruct((_NC * n_pad,), jnp.float32),
      ],
      mesh=mesh,
      scratch_types=[
          pltpu.VMEM((2, _C, _B), jnp.int32),
          pltpu.VMEM((2, _C, _B), jnp.int32),
          pltpu.VMEM((_B, d), jnp.float32),
          pltpu.VMEM((_B, d), jnp.float32),
          pltpu.VMEM((_B,), jnp.float32),
          pltpu.VMEM((rows_per_tile,), jnp.float32),
          pltpu.VMEM_SHARED((n_pad, d), jnp.float32),
          pltpu.VMEM_SHARED((n_pad,), jnp.float32),
          pltpu.SemaphoreType.DMA,
          pltpu.SemaphoreType.DMA,
          pltpu.SemaphoreType.DMA,
          pltpu.SemaphoreType.DMA,
          pltpu.SemaphoreType.DMA,
      ],
  )
  return kern(x, src2, dst2)


def _dense_body(sum_ref, cnt_ref, x_ref, wl_ref, bl_ref, wr_ref, g_ref, b_ref,
                o_ref):
  s = sum_ref[0] + sum_ref[1]
  c = cnt_ref[0] + cnt_ref[1]
  aggr = s / jnp.maximum(c, 1.0)
  xb = x_ref[...]
  f = (lax.dot_general(aggr, wl_ref[...], (((1,), (1,)), ((), ())),
                       preferred_element_type=jnp.float32)
       + lax.dot_general(xb, wr_ref[...], (((1,), (1,)), ((), ())),
                         preferred_element_type=jnp.float32)
       + bl_ref[...])
  f = 0.5 * f * (1.0 + lax.erf(f * (2.0 ** -0.5)))
  mu = jnp.mean(f, axis=-1, keepdims=True)
  zc = f - mu
  var = jnp.mean(zc * zc, axis=-1, keepdims=True)
  o_ref[...] = zc * lax.rsqrt(var + 1e-5) * g_ref[...] + b_ref[...] + xb


def _dense(sums, cnt3, x, W_l, b_l, W_r, gamma, beta):
  n, d = x.shape
  grid = (n // _R,)
  return pl.pallas_call(
      _dense_body,
      grid=grid,
      in_specs=[
          pl.BlockSpec((_NC, _R, d), lambda i: (0, i, 0)),
          pl.BlockSpec((_NC, _R, 1), lambda i: (0, i, 0)),
          pl.BlockSpec((_R, d), lambda i: (i, 0)),
          pl.BlockSpec((d, d), lambda i: (0, 0)),
          pl.BlockSpec((1, d), lambda i: (0, 0)),
          pl.BlockSpec((d, d), lambda i: (0, 0)),
          pl.BlockSpec((1, d), lambda i: (0, 0)),
          pl.BlockSpec((1, d), lambda i: (0, 0)),
      ],
      out_specs=pl.BlockSpec((_R, d), lambda i: (i, 0)),
      out_shape=jax.ShapeDtypeStruct((n, d), jnp.float32),
  )(sums, cnt3, x, W_l, b_l.reshape(1, d), W_r, gamma.reshape(1, d),
    beta.reshape(1, d))


def kernel(x, edge_index, W_l, b_l, W_r, gamma, beta):
  n, d = x.shape
  e = edge_index.shape[1]
  n_pad = _round_up(n + 1, _NS * _B)
  e_pad = _round_up(e, _NC * _NS * _B * _C)
  pad = e_pad - e
  src = jnp.concatenate([edge_index[0], jnp.zeros((pad,), jnp.int32)])
  dst = jnp.concatenate(
      [edge_index[1], jnp.full((pad,), n_pad - 1, jnp.int32)])
  src2 = src.reshape(e_pad // _B, _B)
  dst2 = dst.reshape(e_pad // _B, _B)
  sums, cnts = _sage_aggregate(x, src2, dst2, n_pad)
  cnt3 = cnts.reshape(_NC, n_pad, 1)
  return _dense(sums, cnt3, x, W_l, b_l, W_r, gamma, beta)


# speedup vs baseline: 1.0174x; 1.0174x over previous
import jax
import jax.numpy as jnp
from jax import lax
from jax.experimental import pallas as pl
from jax.experimental.pallas import tpu as pltpu
from jax.experimental.pallas import tpu_sc as plsc

_NC = 2
_NS = 16
_B = 128
_R = 400


def _round_up(v, m):
  return (v + m - 1) // m * m


_NSLOT = 2
_C = 8


def _pad_edges(edge3, e_pad, dump):
  _, eb, b = edge3.shape
  ebp = e_pad // b

  def body(e_ref, o_ref):
    o_ref[:, :eb, :] = e_ref[...]
    o_ref[0:1, eb:, :] = jnp.zeros((1, ebp - eb, b), jnp.int32)
    o_ref[1:2, eb:, :] = jnp.full((1, ebp - eb, b), dump, jnp.int32)

  return pl.pallas_call(
      body,
      out_shape=jax.ShapeDtypeStruct((2, ebp, b), jnp.int32),
  )(edge3)


def _sage_aggregate(x, edge2, n_pad):
  n, d = x.shape
  nb_total = edge2.shape[1]
  nw = _NC * _NS
  nb = nb_total // nw
  rows_per_tile = n_pad // _NS
  nchunk = nb // _C
  assert nb_total == nw * nb and nb % _C == 0 and nchunk >= 3 and nb % 2 == 0
  assert rows_per_tile % _B == 0 and _C % 8 == 0

  mesh = plsc.VectorSubcoreMesh(core_axis_name="c", subcore_axis_name="s")

  def body(x_hbm, edge_hbm, sum_hbm, cnt_hbm,
           sv, dv, r0, r1, ones_v, zrow_v, acc_sh, cnt_sh,
           g0, g1, s0, s1, sc, si):
    rows = (r0, r1)
    sem_g = (g0, g1)
    sem_s = (s0, s1)
    c = lax.axis_index("c")
    s = lax.axis_index("s")
    w = c * _NS + s
    row0 = s * rows_per_tile
    tile_b0 = w * nb

    @pl.loop(0, _B)
    def _(r):
      @pl.loop(0, d, step=16)
      def _(k):
        r0[r, pl.ds(k, 16)] = jnp.zeros((16,), jnp.float32)

    @pl.loop(0, rows_per_tile, step=16)
    def _(i):
      zrow_v[pl.ds(i, 16)] = jnp.zeros((16,), jnp.float32)

    @pl.loop(0, _B, step=16)
    def _(k):
      ones_v[pl.ds(k, 16)] = jnp.ones((16,), jnp.float32)

    def load_chunk(cc):
      pltpu.sync_copy(edge_hbm.at[0, pl.ds(tile_b0 + cc * _C, _C)],
                      sv.at[cc % 2])
      pltpu.sync_copy(edge_hbm.at[1, pl.ds(tile_b0 + cc * _C, _C)],
                      dv.at[cc % 2])

    def fire_chunk(cc):
      pltpu.async_copy(edge_hbm.at[0, pl.ds(tile_b0 + cc * _C, _C)],
                       sv.at[cc % 2], si)
      pltpu.async_copy(edge_hbm.at[1, pl.ds(tile_b0 + cc * _C, _C)],
                       dv.at[cc % 2], si)

    def wait_chunk():
      pltpu.make_async_copy(edge_hbm.at[0, pl.ds(tile_b0, _C)],
                            sv.at[0], si).wait()
      pltpu.make_async_copy(edge_hbm.at[1, pl.ds(tile_b0, _C)],
                            dv.at[0], si).wait()

    def fire_cnt(cc):
      for j in range(_C):
        pltpu.async_copy(ones_v, cnt_sh.at[dv.at[cc % 2, j]], sc, add=True)

    def wait_cnt():
      for _j in range(_C):
        pltpu.make_async_copy(ones_v, cnt_sh.at[dv.at[0, 0]], sc).wait()

    load_chunk(0)
    load_chunk(1)

    for j in range(rows_per_tile // _B):
      pltpu.sync_copy(r0, acc_sh.at[pl.ds(row0 + j * _B, _B)])
    pltpu.sync_copy(zrow_v, cnt_sh.at[pl.ds(row0, rows_per_tile)])
    plsc.subcore_barrier()

    fire_cnt(0)

    def idx(buf, b):
      return buf.at[(b // _C) % 2, b % _C]

    def fire_g(j, b):
      pltpu.async_copy(x_hbm.at[idx(sv, b)], rows[j], sem_g[j])

    def wait_g(j):
      pltpu.make_async_copy(x_hbm.at[sv.at[0, 0]], rows[j], sem_g[j]).wait()

    def fire_s(j, b):
      pltpu.async_copy(rows[j], acc_sh.at[idx(dv, b)], sem_s[j], add=True)

    def wait_s(j):
      pltpu.make_async_copy(rows[j], acc_sh.at[dv.at[0, 0]], sem_s[j]).wait()

    fire_g(0, 0)
    fire_g(1, 1)
    wait_g(0)
    fire_s(0, 0)
    wait_g(1)
    fire_s(1, 1)
    wait_s(0)
    fire_g(0, 2)

    @pl.loop(1, nb // 2 - 1)
    def _(k):
      b = 2 * k
      wait_g(0)
      fire_s(0, b)
      wait_s(1)
      @pl.when(b % _C == 0)
      def _():
        wait_cnt()
        @pl.when(b + _C < nb)
        def _():
          fire_chunk(b // _C + 1)
        fire_cnt(b // _C)
      fire_g(1, b + 1)
      b = 2 * k + 1
      wait_g(1)
      fire_s(1, b)
      wait_s(0)
      @pl.when(jnp.logical_and((b + 1) % _C == 0,
                               jnp.logical_and(b + 1 >= 2 * _C, b + 1 < nb)))
      def _():
        wait_chunk()
      fire_g(0, b + 1)

    wait_g(0)
    fire_s(0, nb - 2)
    wait_s(1)
    fire_g(1, nb - 1)
    wait_g(1)
    fire_s(1, nb - 1)
    wait_s(0)
    wait_s(1)
    wait_cnt()

    plsc.subcore_barrier()

    pltpu.sync_copy(acc_sh.at[pl.ds(row0, rows_per_tile)],
                    sum_hbm.at[c, pl.ds(row0, rows_per_tile)])
    pltpu.sync_copy(cnt_sh.at[pl.ds(row0, rows_per_tile)],
                    cnt_hbm.at[pl.ds(c * n_pad + row0, rows_per_tile)])

  kern = pl.kernel(
      body,
      out_type=[
          jax.ShapeDtypeStruct((_NC, n_pad, d), jnp.float32),
          jax.ShapeDtypeStruct((_NC * n_pad,), jnp.float32),
      ],
      mesh=mesh,
      scratch_types=[
          pltpu.VMEM((2, _C, _B), jnp.int32),
          pltpu.VMEM((2, _C, _B), jnp.int32),
          pltpu.VMEM((_B, d), jnp.float32),
          pltpu.VMEM((_B, d), jnp.float32),
          pltpu.VMEM((_B,), jnp.float32),
          pltpu.VMEM((rows_per_tile,), jnp.float32),
          pltpu.VMEM_SHARED((n_pad, d), jnp.float32),
          pltpu.VMEM_SHARED((n_pad,), jnp.float32),
          pltpu.SemaphoreType.DMA,
          pltpu.SemaphoreType.DMA,
          pltpu.SemaphoreType.DMA,
          pltpu.SemaphoreType.DMA,
          pltpu.SemaphoreType.DMA,
          pltpu.SemaphoreType.DMA,
      ],
  )
  return kern(x, edge2)


def _dense_body(sum_ref, cnt_ref, x_ref, wl_ref, bl_ref, wr_ref, g_ref, b_ref,
                o_ref):
  s = sum_ref[0] + sum_ref[1]
  c = cnt_ref[0] + cnt_ref[1]
  aggr = s / jnp.maximum(c, 1.0)
  xb = x_ref[...]
  f = (lax.dot_general(aggr, wl_ref[...], (((1,), (1,)), ((), ())),
                       preferred_element_type=jnp.float32)
       + lax.dot_general(xb, wr_ref[...], (((1,), (1,)), ((), ())),
                         preferred_element_type=jnp.float32)
       + bl_ref[...])
  f = 0.5 * f * (1.0 + lax.erf(f * (2.0 ** -0.5)))
  mu = jnp.mean(f, axis=-1, keepdims=True)
  zc = f - mu
  var = jnp.mean(zc * zc, axis=-1, keepdims=True)
  o_ref[...] = zc * lax.rsqrt(var + 1e-5) * g_ref[...] + b_ref[...] + xb


def _dense(sums, cnt3, x, W_l, b_l, W_r, gamma, beta):
  n, d = x.shape
  grid = (n // _R,)
  return pl.pallas_call(
      _dense_body,
      grid=grid,
      in_specs=[
          pl.BlockSpec((_NC, _R, d), lambda i: (0, i, 0)),
          pl.BlockSpec((_NC, _R, 1), lambda i: (0, i, 0)),
          pl.BlockSpec((_R, d), lambda i: (i, 0)),
          pl.BlockSpec((d, d), lambda i: (0, 0)),
          pl.BlockSpec((1, d), lambda i: (0, 0)),
          pl.BlockSpec((d, d), lambda i: (0, 0)),
          pl.BlockSpec((1, d), lambda i: (0, 0)),
          pl.BlockSpec((1, d), lambda i: (0, 0)),
      ],
      out_specs=pl.BlockSpec((_R, d), lambda i: (i, 0)),
      out_shape=jax.ShapeDtypeStruct((n, d), jnp.float32),
  )(sums, cnt3, x, W_l, b_l.reshape(1, d), W_r, gamma.reshape(1, d),
    beta.reshape(1, d))


def kernel(x, edge_index, W_l, b_l, W_r, gamma, beta):
  n, d = x.shape
  e = edge_index.shape[1]
  n_pad = _round_up(n + 1, _NS * _B)
  e_pad = _round_up(e, _NC * _NS * _B * _C)
  assert e % _B == 0
  edge3 = edge_index.reshape(2, e // _B, _B)
  edge2 = _pad_edges(edge3, e_pad, n_pad - 1)
  sums, cnts = _sage_aggregate(x, edge2, n_pad)
  cnt3 = cnts.reshape(_NC, n_pad, 1)
  return _dense(sums, cnt3, x, W_l, b_l, W_r, gamma, beta)
